# baseline (speedup 1.0000x reference)
import functools

import jax
import jax.numpy as jnp
from jax.experimental import pallas as pl

N = 100000
D = 128
HEADS = 4
K = 50000

ROWS = 800
GRID = N // ROWS


def _scores_body(x_ref, w1_ref, b1_ref, w2_ref, b2_ref, out_ref):
    xb = x_ref[...]
    h = jnp.dot(xb, w1_ref[...]) + b1_ref[...]
    h = jnp.where(h >= 0, h, 0.2 * h)
    attn2 = jnp.dot(h, w2_ref[...]) + b2_ref[...]
    a0, a1, a2, a3 = (attn2[:, i] for i in range(HEADS))
    attn = ((a0 + a2) + (a1 + a3)) * 0.25
    out_ref[...] = jax.nn.sigmoid(attn)[None, None, :]


def _scores(x, W1, b1, W2, b2):
    out = pl.pallas_call(
        _scores_body,
        grid=(GRID,),
        in_specs=[
            pl.BlockSpec((ROWS, D), lambda i: (i, 0)),
            pl.BlockSpec((D, HEADS * D), lambda i: (0, 0)),
            pl.BlockSpec((HEADS * D,), lambda i: (0,)),
            pl.BlockSpec((HEADS * D, HEADS), lambda i: (0, 0)),
            pl.BlockSpec((HEADS,), lambda i: (0,)),
        ],
        out_specs=pl.BlockSpec((1, 1, ROWS), lambda i: (i, 0, 0)),
        out_shape=jax.ShapeDtypeStruct((GRID, 1, ROWS), jnp.float32),
    )(x, W1, b1, W2, b2)
    return out.reshape(N)


def kernel(x, W1, b1, W2, b2):
    scores = _scores(x, W1, b1, W2, b2)
    top_scores, idx = jax.lax.top_k(scores, K)
    node_feat = jnp.take(x, idx, axis=0)
    scaled_feat = node_feat * (1.0 + top_scores[:, None])
    return scaled_feat, idx, scores


# speedup vs baseline: 1.0390x; 1.0390x over previous
import functools

import jax
import jax.numpy as jnp
from jax import lax
from jax.experimental import pallas as pl
from jax.experimental.pallas import tpu as pltpu
from jax.experimental.pallas import tpu_sc as plsc

N = 100000
D = 128
HEADS = 4
K = 50000

ROWS = 2000
GRID = N // ROWS


def _scores_body(x_ref, w1_ref, b1_ref, w2_ref, b2_ref, out_ref, key_ref):
    xb = x_ref[...]
    h = jnp.dot(xb, w1_ref[...]) + b1_ref[...]
    h = jnp.where(h >= 0, h, 0.2 * h)
    attn2 = jnp.dot(h, w2_ref[...]) + b2_ref[...]
    a0, a1, a2, a3 = (attn2[:, i] for i in range(HEADS))
    attn = ((a0 + a2) + (a1 + a3)) * 0.25
    sc = jax.nn.sigmoid(attn)
    out_ref[...] = sc[None, None, :]
    key_ref[...] = (KEY_TOP
                    - lax.bitcast_convert_type(sc, jnp.int32))[None, None, :]


def _scores(x, W1, b1, W2, b2):
    out = pl.pallas_call(
        _scores_body,
        grid=(GRID,),
        in_specs=[
            pl.BlockSpec((ROWS, D), lambda i: (i, 0)),
            pl.BlockSpec((D, HEADS * D), lambda i: (0, 0)),
            pl.BlockSpec((HEADS * D,), lambda i: (0,)),
            pl.BlockSpec((HEADS * D, HEADS), lambda i: (0, 0)),
            pl.BlockSpec((HEADS,), lambda i: (0,)),
        ],
        out_specs=[pl.BlockSpec((1, 1, ROWS), lambda i: (i, 0, 0)),
                   pl.BlockSpec((1, 1, ROWS), lambda i: (i, 0, 0))],
        out_shape=[jax.ShapeDtypeStruct((GRID, 1, ROWS), jnp.float32),
                   jax.ShapeDtypeStruct((GRID, 1, ROWS), jnp.int32)],
    )(x, W1, b1, W2, b2)
    return out[0].reshape(N), out[1].reshape(N)


NPAD = 100352
CT = NPAD // 16
SL = CT // 16
BINS = 256
PASSES = (0, 8, 16, 24)
KEY_TOP = 0x3F800000
PAD_KEY = 0x7FFFFFFF
OT = 3136


def _sort_body(keys_hbm, idx_out, ks_out, keys_v, vals_v, pos_v,
               hist_v, bcum_v, rcnt_v, pref_v, gcopy_v, gshared,
               kbuf0, vbuf0, kbuf1, vbuf1):
    c = lax.axis_index("c")
    t = lax.axis_index("s")
    lanes = lax.iota(jnp.int32, 16)
    l392 = lanes * SL
    ones = jnp.ones((16,), jnp.int32)

    @pl.when(c == 0)
    def _run():
        base = t * CT

        @pl.when(t < 15)
        def _():
            pltpu.sync_copy(keys_hbm.at[pl.ds(base, CT)], keys_v)

        @pl.when(t == 15)
        def _():
            pltpu.sync_copy(keys_hbm.at[pl.ds(15 * CT, N - 15 * CT)],
                            keys_v.at[pl.ds(0, N - 15 * CT)])

            @pl.loop((N - 15 * CT) // 16, SL)
            def _pad(r):
                keys_v[pl.ds(r * 16, 16)] = jnp.full((16,), PAD_KEY, jnp.int32)

        @pl.loop(0, SL)
        def _init(r):
            o = r * 16
            vals_v[pl.ds(o, 16)] = base + o + lanes

        kbufs = (kbuf0, kbuf1)
        vbufs = (vbuf0, vbuf1)
        for p, shift in enumerate(PASSES):
            if p > 0:
                pltpu.sync_copy(kbufs[(p - 1) % 2].at[pl.ds(base, CT)], keys_v)
                pltpu.sync_copy(vbufs[(p - 1) % 2].at[pl.ds(base, CT)], vals_v)

            zero16 = jnp.zeros((16,), jnp.int32)

            @pl.loop(0, BINS, unroll=4)
            def _zh(i):
                hist_v[pl.ds(i * 16, 16)] = zero16
                rcnt_v[pl.ds(i * 16, 16)] = zero16

            @pl.loop(0, SL)
            def _hist(r):
                k = plsc.load_gather(keys_v, [l392 + r])
                d = jnp.right_shift(k, shift) & (BINS - 1)
                plsc.addupdate_scatter(hist_v, [lanes * BINS + d], ones)

            pltpu.sync_copy(hist_v, gshared.at[pl.ds(t * 4096, 4096)])
            plsc.subcore_barrier()
            pltpu.sync_copy(gshared, gcopy_v)

            totals = []
            for g in range(16):
                def _scan(cc, acc, g=g):
                    v = gcopy_v[pl.ds(cc * BINS + g * 16, 16)]

                    @pl.when((cc >= t * 16) & (cc < t * 16 + 16))
                    def _():
                        plsc.store_scatter(
                            bcum_v, [g * 256 + lanes * 16 + (cc - t * 16)],
                            acc)

                    return acc + v

                acc = lax.fori_loop(0, 256, _scan,
                                    jnp.zeros((16,), jnp.int32))
                totals.append(acc)

            carry = jnp.int32(0)
            for g in range(16):
                cs = plsc.cumsum(totals[g])
                pref_v[pl.ds(g * 16, 16)] = (cs - totals[g]) + carry
                carry = carry + jnp.sum(totals[g])

            @pl.loop(0, SL)
            def _rank(r):
                idxv = l392 + r
                k = plsc.load_gather(keys_v, [idxv])
                d = jnp.right_shift(k, shift) & (BINS - 1)
                dl = d * 16 + lanes
                b = plsc.load_gather(bcum_v, [dl])
                pf = plsc.load_gather(pref_v, [d])
                rc = plsc.load_gather(rcnt_v, [dl])
                plsc.addupdate_scatter(rcnt_v, [dl], ones)
                plsc.store_scatter(pos_v, [idxv], b + pf + rc)

            pltpu.sync_copy(keys_v, kbufs[p % 2].at[pos_v])
            pltpu.sync_copy(vals_v, vbufs[p % 2].at[pos_v])
            plsc.subcore_barrier()

        kb_f = kbufs[(len(PASSES) - 1) % 2]
        vb_f = vbufs[(len(PASSES) - 1) % 2]
        start = t * OT

        def _emit(nt):
            pltpu.sync_copy(kb_f.at[pl.ds(start, nt)],
                            keys_v.at[pl.ds(0, nt)])
            pltpu.sync_copy(keys_v.at[pl.ds(0, nt)],
                            ks_out.at[pl.ds(start, nt)])
            pltpu.sync_copy(vb_f.at[pl.ds(start, nt)],
                            vals_v.at[pl.ds(0, nt)])
            pltpu.sync_copy(vals_v.at[pl.ds(0, nt)],
                            idx_out.at[pl.ds(start, nt)])

        @pl.when(t < 15)
        def _():
            _emit(OT)

        @pl.when(t == 15)
        def _():
            _emit(K - 15 * OT)


def _sort(keys):
    return pl.kernel(
        _sort_body,
        out_type=(
            jax.ShapeDtypeStruct((K,), jnp.int32),
            jax.ShapeDtypeStruct((K,), jnp.int32),
        ),
        mesh=plsc.VectorSubcoreMesh(core_axis_name="c", subcore_axis_name="s"),
        compiler_params=pltpu.CompilerParams(needs_layout_passes=False),
        scratch_types=(
            pltpu.VMEM((CT,), jnp.int32),
            pltpu.VMEM((CT,), jnp.int32),
            pltpu.VMEM((CT,), jnp.int32),
            pltpu.VMEM((16 * BINS,), jnp.int32),
            pltpu.VMEM((16 * BINS,), jnp.int32),
            pltpu.VMEM((16 * BINS,), jnp.int32),
            pltpu.VMEM((BINS,), jnp.int32),
            pltpu.VMEM((16 * 16 * BINS,), jnp.int32),
            pltpu.VMEM_SHARED((16 * 16 * BINS,), jnp.int32),
            pltpu.VMEM_SHARED((NPAD,), jnp.int32),
            pltpu.VMEM_SHARED((NPAD,), jnp.int32),
            pltpu.VMEM_SHARED((NPAD,), jnp.int32),
            pltpu.VMEM_SHARED((NPAD,), jnp.int32),
        ),
    )(keys)


RPW = 1568
WIN = 392


def _gather_body(x_hbm, idx_hbm, ts_hbm, out_hbm, idxb, scb, rowb):
    c = lax.axis_index("c")
    s = lax.axis_index("s")
    w = s * 2 + c
    base = w * RPW

    def _windows(nrows):
        nwin = (nrows + WIN - 1) // WIN
        pltpu.sync_copy(idx_hbm.at[pl.ds(base, nrows)],
                        idxb.at[pl.ds(0, nrows)])
        pltpu.sync_copy(ts_hbm.at[pl.ds(base, nrows)],
                        scb.at[pl.ds(0, nrows)])
        for k in range(nwin):
            wn = min(WIN, nrows - k * WIN)
            pltpu.sync_copy(x_hbm.at[idxb.at[pl.ds(k * WIN, wn)]],
                            rowb.at[pl.ds(0, wn)])

            @pl.loop(0, wn)
            def _scale(r):
                sv = 1.0 + plsc.load_gather(
                    scb, [jnp.full((16,), k * WIN + r, jnp.int32)])
                for u in range(D // 16):
                    rowb[r, pl.ds(u * 16, 16)] = rowb[r, pl.ds(u * 16, 16)] * sv

            pltpu.sync_copy(rowb.at[pl.ds(0, wn)],
                            out_hbm.at[pl.ds(base + k * WIN, wn)])

    @pl.when(w < 31)
    def _():
        _windows(RPW)

    @pl.when(w == 31)
    def _():
        _windows(K - 31 * RPW)


def _gather_scale(x, idx, top_scores):
    return pl.kernel(
        _gather_body,
        out_type=jax.ShapeDtypeStruct((K, D), jnp.float32),
        mesh=plsc.VectorSubcoreMesh(core_axis_name="c", subcore_axis_name="s"),
        compiler_params=pltpu.CompilerParams(needs_layout_passes=False),
        scratch_types=(
            pltpu.VMEM((RPW,), jnp.int32),
            pltpu.VMEM((RPW,), jnp.float32),
            pltpu.VMEM((WIN, D), jnp.float32),
        ),
    )(x, idx, top_scores)


def kernel(x, W1, b1, W2, b2):
    scores, keys = _scores(x, W1, b1, W2, b2)
    idx, keys_sorted = _sort(keys)
    top_scores = lax.bitcast_convert_type(KEY_TOP - keys_sorted, jnp.float32)
    scaled_feat = _gather_scale(x, idx, top_scores)
    return scaled_feat, idx, scores


# speedup vs baseline: 1.2257x; 1.1797x over previous
import functools

import jax
import jax.numpy as jnp
from jax import lax
from jax.experimental import pallas as pl
from jax.experimental.pallas import tpu as pltpu
from jax.experimental.pallas import tpu_sc as plsc

N = 100000
D = 128
HEADS = 4
K = 50000

ROWS = 2000
GRID = N // ROWS


def _scores_body(x_ref, w1_ref, b1_ref, w2_ref, b2_ref, out_ref, key_ref):
    xb = x_ref[...]
    h = jnp.dot(xb, w1_ref[...]) + b1_ref[...]
    h = jnp.where(h >= 0, h, 0.2 * h)
    attn2 = jnp.dot(h, w2_ref[...]) + b2_ref[...]
    a0, a1, a2, a3 = (attn2[:, i] for i in range(HEADS))
    attn = ((a0 + a2) + (a1 + a3)) * 0.25
    sc = jax.nn.sigmoid(attn)
    out_ref[...] = sc[None, None, :]
    key_ref[...] = (KEY_TOP
                    - lax.bitcast_convert_type(sc, jnp.int32))[None, None, :]


def _scores(x, W1, b1, W2, b2):
    out = pl.pallas_call(
        _scores_body,
        grid=(GRID,),
        in_specs=[
            pl.BlockSpec((ROWS, D), lambda i: (i, 0)),
            pl.BlockSpec((D, HEADS * D), lambda i: (0, 0)),
            pl.BlockSpec((HEADS * D,), lambda i: (0,)),
            pl.BlockSpec((HEADS * D, HEADS), lambda i: (0, 0)),
            pl.BlockSpec((HEADS,), lambda i: (0,)),
        ],
        out_specs=[pl.BlockSpec((1, 1, ROWS), lambda i: (i, 0, 0)),
                   pl.BlockSpec((1, 1, ROWS), lambda i: (i, 0, 0))],
        out_shape=[jax.ShapeDtypeStruct((GRID, 1, ROWS), jnp.float32),
                   jax.ShapeDtypeStruct((GRID, 1, ROWS), jnp.int32)],
    )(x, W1, b1, W2, b2)
    return out[0].reshape(N), out[1].reshape(N)


NPAD = 100352
CT = NPAD // 16
SL = CT // 16
BINS = 1024
NG = BINS // 16
PASSES = (0, 10, 20)
KEY_TOP = 0x3F800000
PAD_KEY = 0x7FFFFFFF
OT = 3136


def _sort_body(keys_hbm, idx_out, ks_out, keys_v, vals_v, pos_v,
               hist_v, rcnt_v, prior_v, totv_v, tsum_v, tcopy_v, tshared,
               kbuf0, vbuf0, kbuf1, vbuf1):
    c = lax.axis_index("c")
    t = lax.axis_index("s")
    lanes = lax.iota(jnp.int32, 16)
    l392 = lanes * SL
    ones = jnp.ones((16,), jnp.int32)

    @pl.when(c == 0)
    def _run():
        base = t * CT

        @pl.when(t < 15)
        def _():
            pltpu.sync_copy(keys_hbm.at[pl.ds(base, CT)], keys_v)

        @pl.when(t == 15)
        def _():
            pltpu.sync_copy(keys_hbm.at[pl.ds(15 * CT, N - 15 * CT)],
                            keys_v.at[pl.ds(0, N - 15 * CT)])

            @pl.loop((N - 15 * CT) // 16, SL)
            def _pad(r):
                keys_v[pl.ds(r * 16, 16)] = jnp.full((16,), PAD_KEY, jnp.int32)

        @pl.loop(0, SL)
        def _init(r):
            o = r * 16
            vals_v[pl.ds(o, 16)] = base + o + lanes

        kbufs = (kbuf0, kbuf1)
        vbufs = (vbuf0, vbuf1)
        for p, shift in enumerate(PASSES):
            if p > 0:
                pltpu.sync_copy(kbufs[(p - 1) % 2].at[pl.ds(base, CT)], keys_v)
                pltpu.sync_copy(vbufs[(p - 1) % 2].at[pl.ds(base, CT)], vals_v)

            zero16 = jnp.zeros((16,), jnp.int32)

            @pl.loop(0, BINS, unroll=4)
            def _zh(i):
                hist_v[pl.ds(i * 16, 16)] = zero16

            @pl.loop(0, SL, unroll=2)
            def _hist(r):
                k = plsc.load_gather(keys_v, [l392 + r])
                d = jnp.right_shift(k, shift) & (BINS - 1)
                plsc.addupdate_scatter(hist_v, [lanes * BINS + d], ones)

            @pl.loop(0, NG)
            def _tsum(g):
                acc = hist_v[pl.ds(g * 16, 16)]
                for l in range(1, 16):
                    acc = acc + hist_v[pl.ds(l * BINS + g * 16, 16)]
                tsum_v[pl.ds(g * 16, 16)] = acc

            pltpu.sync_copy(tsum_v, tshared.at[pl.ds(t * BINS, BINS)])
            plsc.subcore_barrier()
            pltpu.sync_copy(tshared, tcopy_v)

            @pl.loop(0, NG)
            def _prior(g):
                acc = zero16
                for tt in range(16):
                    @pl.when(t == tt)
                    def _(acc=acc):
                        prior_v[pl.ds(g * 16, 16)] = acc
                    acc = acc + tcopy_v[pl.ds(tt * BINS + g * 16, 16)]
                totv_v[pl.ds(g * 16, 16)] = acc

            def _pref(g, carry):
                tot = totv_v[pl.ds(g * 16, 16)]
                excl = (plsc.cumsum(tot) - tot) + carry
                prior_v[pl.ds(g * 16, 16)] = prior_v[pl.ds(g * 16, 16)] + excl
                return carry + jnp.sum(tot)

            lax.fori_loop(0, NG, _pref, jnp.int32(0))

            @pl.loop(0, NG)
            def _seed(g):
                run = prior_v[pl.ds(g * 16, 16)]
                for l in range(16):
                    plsc.store_scatter(
                        rcnt_v, [(g * 16 + lanes) * 16 + l], run)
                    run = run + hist_v[pl.ds(l * BINS + g * 16, 16)]

            @pl.loop(0, SL, unroll=2)
            def _rank(r):
                idxv = l392 + r
                k = plsc.load_gather(keys_v, [idxv])
                d = jnp.right_shift(k, shift) & (BINS - 1)
                dl = d * 16 + lanes
                rc = plsc.load_gather(rcnt_v, [dl])
                plsc.addupdate_scatter(rcnt_v, [dl], ones)
                plsc.store_scatter(pos_v, [idxv], rc)

            pltpu.sync_copy(keys_v, kbufs[p % 2].at[pos_v])
            pltpu.sync_copy(vals_v, vbufs[p % 2].at[pos_v])
            plsc.subcore_barrier()

        kb_f = kbufs[(len(PASSES) - 1) % 2]
        vb_f = vbufs[(len(PASSES) - 1) % 2]
        start = t * OT

        def _emit(nt):
            pltpu.sync_copy(kb_f.at[pl.ds(start, nt)],
                            keys_v.at[pl.ds(0, nt)])
            pltpu.sync_copy(keys_v.at[pl.ds(0, nt)],
                            ks_out.at[pl.ds(start, nt)])
            pltpu.sync_copy(vb_f.at[pl.ds(start, nt)],
                            vals_v.at[pl.ds(0, nt)])
            pltpu.sync_copy(vals_v.at[pl.ds(0, nt)],
                            idx_out.at[pl.ds(start, nt)])

        @pl.when(t < 15)
        def _():
            _emit(OT)

        @pl.when(t == 15)
        def _():
            _emit(K - 15 * OT)


def _sort(keys):
    return pl.kernel(
        _sort_body,
        out_type=(
            jax.ShapeDtypeStruct((K,), jnp.int32),
            jax.ShapeDtypeStruct((K,), jnp.int32),
        ),
        mesh=plsc.VectorSubcoreMesh(core_axis_name="c", subcore_axis_name="s"),
        compiler_params=pltpu.CompilerParams(needs_layout_passes=False),
        scratch_types=(
            pltpu.VMEM((CT,), jnp.int32),
            pltpu.VMEM((CT,), jnp.int32),
            pltpu.VMEM((CT,), jnp.int32),
            pltpu.VMEM((16 * BINS,), jnp.int32),
            pltpu.VMEM((16 * BINS,), jnp.int32),
            pltpu.VMEM((BINS,), jnp.int32),
            pltpu.VMEM((BINS,), jnp.int32),
            pltpu.VMEM((BINS,), jnp.int32),
            pltpu.VMEM((16 * BINS,), jnp.int32),
            pltpu.VMEM_SHARED((16 * BINS,), jnp.int32),
            pltpu.VMEM_SHARED((NPAD,), jnp.int32),
            pltpu.VMEM_SHARED((NPAD,), jnp.int32),
            pltpu.VMEM_SHARED((NPAD,), jnp.int32),
            pltpu.VMEM_SHARED((NPAD,), jnp.int32),
        ),
    )(keys)


RPW = 1568
WIN = 392


def _gather_body(x_hbm, idx_hbm, ts_hbm, out_hbm, idxb, scb, rowb):
    c = lax.axis_index("c")
    s = lax.axis_index("s")
    w = s * 2 + c
    base = w * RPW

    def _windows(nrows):
        nwin = (nrows + WIN - 1) // WIN
        pltpu.sync_copy(idx_hbm.at[pl.ds(base, nrows)],
                        idxb.at[pl.ds(0, nrows)])
        pltpu.sync_copy(ts_hbm.at[pl.ds(base, nrows)],
                        scb.at[pl.ds(0, nrows)])
        for k in range(nwin):
            wn = min(WIN, nrows - k * WIN)
            pltpu.sync_copy(x_hbm.at[idxb.at[pl.ds(k * WIN, wn)]],
                            rowb.at[pl.ds(0, wn)])

            @pl.loop(0, wn)
            def _scale(r):
                sv = 1.0 + plsc.load_gather(
                    scb, [jnp.full((16,), k * WIN + r, jnp.int32)])
                for u in range(D // 16):
                    rowb[r, pl.ds(u * 16, 16)] = rowb[r, pl.ds(u * 16, 16)] * sv

            pltpu.sync_copy(rowb.at[pl.ds(0, wn)],
                            out_hbm.at[pl.ds(base + k * WIN, wn)])

    @pl.when(w < 31)
    def _():
        _windows(RPW)

    @pl.when(w == 31)
    def _():
        _windows(K - 31 * RPW)


def _gather_scale(x, idx, top_scores):
    return pl.kernel(
        _gather_body,
        out_type=jax.ShapeDtypeStruct((K, D), jnp.float32),
        mesh=plsc.VectorSubcoreMesh(core_axis_name="c", subcore_axis_name="s"),
        compiler_params=pltpu.CompilerParams(needs_layout_passes=False),
        scratch_types=(
            pltpu.VMEM((RPW,), jnp.int32),
            pltpu.VMEM((RPW,), jnp.float32),
            pltpu.VMEM((WIN, D), jnp.float32),
        ),
    )(x, idx, top_scores)


def kernel(x, W1, b1, W2, b2):
    scores, keys = _scores(x, W1, b1, W2, b2)
    idx, keys_sorted = _sort(keys)
    top_scores = lax.bitcast_convert_type(KEY_TOP - keys_sorted, jnp.float32)
    scaled_feat = _gather_scale(x, idx, top_scores)
    return scaled_feat, idx, scores


# speedup vs baseline: 1.7127x; 1.3974x over previous
import functools

import jax
import jax.numpy as jnp
from jax import lax
from jax.experimental import pallas as pl
from jax.experimental.pallas import tpu as pltpu
from jax.experimental.pallas import tpu_sc as plsc

N = 100000
D = 128
HEADS = 4
K = 50000

ROWS = 2000
GRID = N // ROWS


def _scores_body(x_ref, w1_ref, b1_ref, w2_ref, b2_ref, out_ref, key_ref):
    xb = x_ref[...]
    h = jnp.dot(xb, w1_ref[...]) + b1_ref[...]
    h = jnp.where(h >= 0, h, 0.2 * h)
    attn2 = lax.dot_general(w2_ref[...], h, (((0,), (1,)), ((), ())))
    attn2 = attn2 + b2_ref[...][:, None]
    attn = ((attn2[0:1, :] + attn2[2:3, :])
            + (attn2[1:2, :] + attn2[3:4, :])) * 0.25
    sc = jax.nn.sigmoid(attn)
    out_ref[...] = sc[None]
    key_ref[...] = (KEY_TOP - lax.bitcast_convert_type(sc, jnp.int32))[None]


def _scores(x, W1, b1, W2, b2):
    out = pl.pallas_call(
        _scores_body,
        grid=(GRID,),
        in_specs=[
            pl.BlockSpec((ROWS, D), lambda i: (i, 0)),
            pl.BlockSpec((D, HEADS * D), lambda i: (0, 0)),
            pl.BlockSpec((HEADS * D,), lambda i: (0,)),
            pl.BlockSpec((HEADS * D, HEADS), lambda i: (0, 0)),
            pl.BlockSpec((HEADS,), lambda i: (0,)),
        ],
        out_specs=[pl.BlockSpec((1, 1, ROWS), lambda i: (i, 0, 0)),
                   pl.BlockSpec((1, 1, ROWS), lambda i: (i, 0, 0))],
        out_shape=[jax.ShapeDtypeStruct((GRID, 1, ROWS), jnp.float32),
                   jax.ShapeDtypeStruct((GRID, 1, ROWS), jnp.int32)],
    )(x, W1, b1, W2, b2)
    return out[0].reshape(N), out[1].reshape(N)


NPAD = 100352
CT = NPAD // 16
SL = CT // 16
BINS = 1024
NG = BINS // 16
PASSES = (0, 10, 20)
KEY_TOP = 0x3F800000
PAD_KEY = 0x7FFFFFFF
OT = 3136


def _sort_body(keys_hbm, idx_out, ks_out, keys_v, vals_v, pos_v,
               hist_v, rcnt_v, prior_v, totv_v, tsum_v, tcopy_v, tshared,
               kbuf0, vbuf0, kbuf1, vbuf1):
    c = lax.axis_index("c")
    t = lax.axis_index("s")
    lanes = lax.iota(jnp.int32, 16)
    l392 = lanes * SL
    ones = jnp.ones((16,), jnp.int32)

    @pl.when(c == 0)
    def _run():
        base = t * CT

        @pl.when(t < 15)
        def _():
            pltpu.sync_copy(keys_hbm.at[pl.ds(base, CT)], keys_v)

        @pl.when(t == 15)
        def _():
            pltpu.sync_copy(keys_hbm.at[pl.ds(15 * CT, N - 15 * CT)],
                            keys_v.at[pl.ds(0, N - 15 * CT)])

            @pl.loop((N - 15 * CT) // 16, SL)
            def _pad(r):
                keys_v[pl.ds(r * 16, 16)] = jnp.full((16,), PAD_KEY, jnp.int32)

        @pl.loop(0, SL)
        def _init(r):
            o = r * 16
            vals_v[pl.ds(o, 16)] = base + o + lanes

        kbufs = (kbuf0, kbuf1)
        vbufs = (vbuf0, vbuf1)
        for p, shift in enumerate(PASSES):
            if p > 0:
                pltpu.sync_copy(kbufs[(p - 1) % 2].at[pl.ds(base, CT)], keys_v)
                pltpu.sync_copy(vbufs[(p - 1) % 2].at[pl.ds(base, CT)], vals_v)

            zero16 = jnp.zeros((16,), jnp.int32)

            @pl.loop(0, BINS, unroll=4)
            def _zh(i):
                hist_v[pl.ds(i * 16, 16)] = zero16

            @pl.loop(0, SL, unroll=2)
            def _hist(r):
                k = plsc.load_gather(keys_v, [l392 + r])
                d = jnp.right_shift(k, shift) & (BINS - 1)
                plsc.addupdate_scatter(hist_v, [lanes * BINS + d], ones)

            @pl.loop(0, NG)
            def _tsum(g):
                acc = hist_v[pl.ds(g * 16, 16)]
                for l in range(1, 16):
                    acc = acc + hist_v[pl.ds(l * BINS + g * 16, 16)]
                tsum_v[pl.ds(g * 16, 16)] = acc

            pltpu.sync_copy(tsum_v, tshared.at[pl.ds(t * BINS, BINS)])
            plsc.subcore_barrier()
            pltpu.sync_copy(tshared, tcopy_v)

            @pl.loop(0, NG)
            def _prior(g):
                acc = zero16
                for tt in range(16):
                    @pl.when(t == tt)
                    def _(acc=acc):
                        prior_v[pl.ds(g * 16, 16)] = acc
                    acc = acc + tcopy_v[pl.ds(tt * BINS + g * 16, 16)]
                totv_v[pl.ds(g * 16, 16)] = acc

            def _pref(g, carry):
                tot = totv_v[pl.ds(g * 16, 16)]
                excl = (plsc.cumsum(tot) - tot) + carry
                prior_v[pl.ds(g * 16, 16)] = prior_v[pl.ds(g * 16, 16)] + excl
                return carry + jnp.sum(tot)

            lax.fori_loop(0, NG, _pref, jnp.int32(0))

            @pl.loop(0, NG)
            def _seed(g):
                run = prior_v[pl.ds(g * 16, 16)]
                for l in range(16):
                    plsc.store_scatter(
                        rcnt_v, [(g * 16 + lanes) * 16 + l], run)
                    run = run + hist_v[pl.ds(l * BINS + g * 16, 16)]

            @pl.loop(0, SL, unroll=2)
            def _rank(r):
                idxv = l392 + r
                k = plsc.load_gather(keys_v, [idxv])
                d = jnp.right_shift(k, shift) & (BINS - 1)
                dl = d * 16 + lanes
                rc = plsc.load_gather(rcnt_v, [dl])
                plsc.addupdate_scatter(rcnt_v, [dl], ones)
                plsc.store_scatter(pos_v, [idxv], rc)

            pltpu.sync_copy(keys_v, kbufs[p % 2].at[pos_v])
            pltpu.sync_copy(vals_v, vbufs[p % 2].at[pos_v])
            plsc.subcore_barrier()

        kb_f = kbufs[(len(PASSES) - 1) % 2]
        vb_f = vbufs[(len(PASSES) - 1) % 2]
        start = t * OT

        def _emit(nt):
            pltpu.sync_copy(kb_f.at[pl.ds(start, nt)],
                            keys_v.at[pl.ds(0, nt)])
            pltpu.sync_copy(keys_v.at[pl.ds(0, nt)],
                            ks_out.at[pl.ds(start, nt)])
            pltpu.sync_copy(vb_f.at[pl.ds(start, nt)],
                            vals_v.at[pl.ds(0, nt)])
            pltpu.sync_copy(vals_v.at[pl.ds(0, nt)],
                            idx_out.at[pl.ds(start, nt)])

        @pl.when(t < 15)
        def _():
            _emit(OT)

        @pl.when(t == 15)
        def _():
            _emit(K - 15 * OT)


def _sort(keys):
    return pl.kernel(
        _sort_body,
        out_type=(
            jax.ShapeDtypeStruct((K,), jnp.int32),
            jax.ShapeDtypeStruct((K,), jnp.int32),
        ),
        mesh=plsc.VectorSubcoreMesh(core_axis_name="c", subcore_axis_name="s"),
        compiler_params=pltpu.CompilerParams(needs_layout_passes=False),
        scratch_types=(
            pltpu.VMEM((CT,), jnp.int32),
            pltpu.VMEM((CT,), jnp.int32),
            pltpu.VMEM((CT,), jnp.int32),
            pltpu.VMEM((16 * BINS,), jnp.int32),
            pltpu.VMEM((16 * BINS,), jnp.int32),
            pltpu.VMEM((BINS,), jnp.int32),
            pltpu.VMEM((BINS,), jnp.int32),
            pltpu.VMEM((BINS,), jnp.int32),
            pltpu.VMEM((16 * BINS,), jnp.int32),
            pltpu.VMEM_SHARED((16 * BINS,), jnp.int32),
            pltpu.VMEM_SHARED((NPAD,), jnp.int32),
            pltpu.VMEM_SHARED((NPAD,), jnp.int32),
            pltpu.VMEM_SHARED((NPAD,), jnp.int32),
            pltpu.VMEM_SHARED((NPAD,), jnp.int32),
        ),
    )(keys)


RPW = 1568
WIN = 392


def _gather_body(x_hbm, idx_hbm, ts_hbm, out_hbm, idxb, scb, rowb):
    c = lax.axis_index("c")
    s = lax.axis_index("s")
    w = s * 2 + c
    base = w * RPW

    def _windows(nrows):
        nwin = (nrows + WIN - 1) // WIN
        pltpu.sync_copy(idx_hbm.at[pl.ds(base, nrows)],
                        idxb.at[pl.ds(0, nrows)])
        pltpu.sync_copy(ts_hbm.at[pl.ds(base, nrows)],
                        scb.at[pl.ds(0, nrows)])
        for k in range(nwin):
            wn = min(WIN, nrows - k * WIN)
            pltpu.sync_copy(x_hbm.at[idxb.at[pl.ds(k * WIN, wn)]],
                            rowb.at[pl.ds(0, wn)])

            @pl.loop(0, wn)
            def _scale(r):
                sv = 1.0 + plsc.load_gather(
                    scb, [jnp.full((16,), k * WIN + r, jnp.int32)])
                for u in range(D // 16):
                    rowb[r, pl.ds(u * 16, 16)] = rowb[r, pl.ds(u * 16, 16)] * sv

            pltpu.sync_copy(rowb.at[pl.ds(0, wn)],
                            out_hbm.at[pl.ds(base + k * WIN, wn)])

    @pl.when(w < 31)
    def _():
        _windows(RPW)

    @pl.when(w == 31)
    def _():
        _windows(K - 31 * RPW)


def _gather_scale(x, idx, top_scores):
    return pl.kernel(
        _gather_body,
        out_type=jax.ShapeDtypeStruct((K, D), jnp.float32),
        mesh=plsc.VectorSubcoreMesh(core_axis_name="c", subcore_axis_name="s"),
        compiler_params=pltpu.CompilerParams(needs_layout_passes=False),
        scratch_types=(
            pltpu.VMEM((RPW,), jnp.int32),
            pltpu.VMEM((RPW,), jnp.float32),
            pltpu.VMEM((WIN, D), jnp.float32),
        ),
    )(x, idx, top_scores)


def kernel(x, W1, b1, W2, b2):
    scores, keys = _scores(x, W1, b1, W2, b2)
    idx, keys_sorted = _sort(keys)
    top_scores = lax.bitcast_convert_type(KEY_TOP - keys_sorted, jnp.float32)
    scaled_feat = _gather_scale(x, idx, top_scores)
    return scaled_feat, idx, scores


# speedup vs baseline: 1.8440x; 1.0766x over previous
import functools

import jax
import jax.numpy as jnp
from jax import lax
from jax.experimental import pallas as pl
from jax.experimental.pallas import tpu as pltpu
from jax.experimental.pallas import tpu_sc as plsc

N = 100000
D = 128
HEADS = 4
K = 50000

ROWS = 2000
GRID = N // ROWS


def _scores_body(x_ref, w1_ref, b1_ref, w2_ref, b2_ref, out_ref, key_ref):
    xb = x_ref[...]
    h = jnp.dot(xb, w1_ref[...]) + b1_ref[...]
    h = jnp.where(h >= 0, h, 0.2 * h)
    attn2 = lax.dot_general(w2_ref[...], h, (((0,), (1,)), ((), ())))
    attn2 = attn2 + b2_ref[...][:, None]
    attn = ((attn2[0:1, :] + attn2[2:3, :])
            + (attn2[1:2, :] + attn2[3:4, :])) * 0.25
    sc = jax.nn.sigmoid(attn)
    out_ref[...] = sc[None]
    key_ref[...] = (KEY_TOP - lax.bitcast_convert_type(sc, jnp.int32))[None]


def _scores(x, W1, b1, W2, b2):
    out = pl.pallas_call(
        _scores_body,
        grid=(GRID,),
        in_specs=[
            pl.BlockSpec((ROWS, D), lambda i: (i, 0)),
            pl.BlockSpec((D, HEADS * D), lambda i: (0, 0)),
            pl.BlockSpec((HEADS * D,), lambda i: (0,)),
            pl.BlockSpec((HEADS * D, HEADS), lambda i: (0, 0)),
            pl.BlockSpec((HEADS,), lambda i: (0,)),
        ],
        out_specs=[pl.BlockSpec((1, 1, ROWS), lambda i: (i, 0, 0)),
                   pl.BlockSpec((1, 1, ROWS), lambda i: (i, 0, 0))],
        out_shape=[jax.ShapeDtypeStruct((GRID, 1, ROWS), jnp.float32),
                   jax.ShapeDtypeStruct((GRID, 1, ROWS), jnp.int32)],
    )(x, W1, b1, W2, b2)
    return out[0].reshape(N), out[1].reshape(N)


NPAD = 100352
CT = NPAD // 16
SL = CT // 16
BINS = 1024
NG = BINS // 16
PASSES = (0, 10, 20)
KEY_TOP = 0x3F800000
PAD_KEY = 0x7FFFFFFF
OT = 3136


def _sort_body(keys_hbm, idx_out, ks_out, keys_v, vals_v, pos_v,
               hist_v, rcnt_v, prior_v, totv_v, tsum_v, tcopy_v, tshared,
               kbuf0, vbuf0, kbuf1, vbuf1, sem_a, sem_b):
    c = lax.axis_index("c")
    t = lax.axis_index("s")
    lanes = lax.iota(jnp.int32, 16)
    l392 = lanes * SL
    ones = jnp.ones((16,), jnp.int32)

    @pl.when(c == 0)
    def _run():
        base = t * CT

        @pl.when(t < 15)
        def _():
            pltpu.sync_copy(keys_hbm.at[pl.ds(base, CT)], keys_v)

        @pl.when(t == 15)
        def _():
            pltpu.sync_copy(keys_hbm.at[pl.ds(15 * CT, N - 15 * CT)],
                            keys_v.at[pl.ds(0, N - 15 * CT)])

            @pl.loop((N - 15 * CT) // 16, SL)
            def _pad(r):
                keys_v[pl.ds(r * 16, 16)] = jnp.full((16,), PAD_KEY, jnp.int32)

        @pl.loop(0, SL, unroll=4)
        def _init(r):
            o = r * 16
            vals_v[pl.ds(o, 16)] = base + o + lanes

        kbufs = (kbuf0, kbuf1)
        vbufs = (vbuf0, vbuf1)
        for p, shift in enumerate(PASSES):
            if p > 0:
                pltpu.sync_copy(kbufs[(p - 1) % 2].at[pl.ds(base, CT)], keys_v)
                pltpu.sync_copy(vbufs[(p - 1) % 2].at[pl.ds(base, CT)], vals_v)

            zero16 = jnp.zeros((16,), jnp.int32)

            @pl.loop(0, BINS, unroll=4)
            def _zh(i):
                hist_v[pl.ds(i * 16, 16)] = zero16

            @pl.loop(0, SL, unroll=4)
            def _hist(r):
                k = plsc.load_gather(keys_v, [l392 + r])
                d = jnp.right_shift(k, shift) & (BINS - 1)
                plsc.addupdate_scatter(hist_v, [lanes * BINS + d], ones)

            @pl.loop(0, NG)
            def _tsum(g):
                acc = hist_v[pl.ds(g * 16, 16)]
                for l in range(1, 16):
                    acc = acc + hist_v[pl.ds(l * BINS + g * 16, 16)]
                tsum_v[pl.ds(g * 16, 16)] = acc

            pltpu.sync_copy(tsum_v, tshared.at[pl.ds(t * BINS, BINS)])
            plsc.subcore_barrier()
            pltpu.sync_copy(tshared, tcopy_v)

            @pl.loop(0, NG)
            def _prior(g):
                acc = zero16
                for tt in range(16):
                    @pl.when(t == tt)
                    def _(acc=acc):
                        prior_v[pl.ds(g * 16, 16)] = acc
                    acc = acc + tcopy_v[pl.ds(tt * BINS + g * 16, 16)]
                totv_v[pl.ds(g * 16, 16)] = acc

            def _pref(g, carry):
                tot = totv_v[pl.ds(g * 16, 16)]
                excl = (plsc.cumsum(tot) - tot) + carry
                prior_v[pl.ds(g * 16, 16)] = prior_v[pl.ds(g * 16, 16)] + excl
                return carry + jnp.sum(tot)

            lax.fori_loop(0, NG, _pref, jnp.int32(0))

            @pl.loop(0, NG)
            def _seed(g):
                run = prior_v[pl.ds(g * 16, 16)]
                for l in range(16):
                    plsc.store_scatter(
                        rcnt_v, [(g * 16 + lanes) * 16 + l], run)
                    run = run + hist_v[pl.ds(l * BINS + g * 16, 16)]

            @pl.loop(0, SL, unroll=4)
            def _rank(r):
                idxv = l392 + r
                k = plsc.load_gather(keys_v, [idxv])
                d = jnp.right_shift(k, shift) & (BINS - 1)
                dl = d * 16 + lanes
                rc = plsc.load_gather(rcnt_v, [dl])
                plsc.addupdate_scatter(rcnt_v, [dl], ones)
                plsc.store_scatter(pos_v, [idxv], rc)

            cp_k = pltpu.async_copy(keys_v, kbufs[p % 2].at[pos_v], sem_a)
            cp_v = pltpu.async_copy(vals_v, vbufs[p % 2].at[pos_v], sem_b)
            cp_k.wait()
            cp_v.wait()
            plsc.subcore_barrier()

        kb_f = kbufs[(len(PASSES) - 1) % 2]
        vb_f = vbufs[(len(PASSES) - 1) % 2]
        start = t * OT

        def _emit(nt):
            cp1 = pltpu.async_copy(kb_f.at[pl.ds(start, nt)],
                                   keys_v.at[pl.ds(0, nt)], sem_a)
            cp2 = pltpu.async_copy(vb_f.at[pl.ds(start, nt)],
                                   vals_v.at[pl.ds(0, nt)], sem_b)
            cp1.wait()
            cp2.wait()
            cp3 = pltpu.async_copy(keys_v.at[pl.ds(0, nt)],
                                   ks_out.at[pl.ds(start, nt)], sem_a)
            cp4 = pltpu.async_copy(vals_v.at[pl.ds(0, nt)],
                                   idx_out.at[pl.ds(start, nt)], sem_b)
            cp3.wait()
            cp4.wait()

        @pl.when(t < 15)
        def _():
            _emit(OT)

        @pl.when(t == 15)
        def _():
            _emit(K - 15 * OT)


def _sort(keys):
    return pl.kernel(
        _sort_body,
        out_type=(
            jax.ShapeDtypeStruct((K,), jnp.int32),
            jax.ShapeDtypeStruct((K,), jnp.int32),
        ),
        mesh=plsc.VectorSubcoreMesh(core_axis_name="c", subcore_axis_name="s"),
        compiler_params=pltpu.CompilerParams(needs_layout_passes=False),
        scratch_types=(
            pltpu.VMEM((CT,), jnp.int32),
            pltpu.VMEM((CT,), jnp.int32),
            pltpu.VMEM((CT,), jnp.int32),
            pltpu.VMEM((16 * BINS,), jnp.int32),
            pltpu.VMEM((16 * BINS,), jnp.int32),
            pltpu.VMEM((BINS,), jnp.int32),
            pltpu.VMEM((BINS,), jnp.int32),
            pltpu.VMEM((BINS,), jnp.int32),
            pltpu.VMEM((16 * BINS,), jnp.int32),
            pltpu.VMEM_SHARED((16 * BINS,), jnp.int32),
            pltpu.VMEM_SHARED((NPAD,), jnp.int32),
            pltpu.VMEM_SHARED((NPAD,), jnp.int32),
            pltpu.VMEM_SHARED((NPAD,), jnp.int32),
            pltpu.VMEM_SHARED((NPAD,), jnp.int32),
            pltpu.SemaphoreType.DMA,
            pltpu.SemaphoreType.DMA,
        ),
    )(keys)


RPW = 1568
WIN = 392


def _gather_body(x_hbm, idx_hbm, ts_hbm, out_hbm, idxb, scb, rowb0, rowb1,
                 sin0, sin1, sout0, sout1):
    c = lax.axis_index("c")
    s = lax.axis_index("s")
    w = s * 2 + c
    base = w * RPW
    bufs = (rowb0, rowb1)
    isems = (sin0, sin1)
    osems = (sout0, sout1)

    def _windows(nrows):
        sizes = []
        left = nrows
        while left > 0:
            sizes.append(min(WIN, left))
            left -= WIN
        nwin = len(sizes)
        pltpu.sync_copy(idx_hbm.at[pl.ds(base, nrows)],
                        idxb.at[pl.ds(0, nrows)])
        pltpu.sync_copy(ts_hbm.at[pl.ds(base, nrows)],
                        scb.at[pl.ds(0, nrows)])

        @pl.loop(0, RPW // 16, unroll=4)
        def _onep(i):
            scb[pl.ds(i * 16, 16)] = 1.0 + scb[pl.ds(i * 16, 16)]

        def _start_in(k):
            wn = sizes[k]
            return pltpu.async_copy(
                x_hbm.at[idxb.at[pl.ds(k * WIN, wn)]],
                bufs[k % 2].at[pl.ds(0, wn)], isems[k % 2])

        din = _start_in(0)
        dout = [None] * nwin
        for k in range(nwin):
            wn = sizes[k]
            nxt = None
            if k + 1 < nwin:
                if k >= 1:
                    dout[k - 1].wait()
                nxt = _start_in(k + 1)
            din.wait()
            rb = bufs[k % 2]

            @pl.loop(0, wn, unroll=2)
            def _scale(r, k=k, rb=rb):
                sv = plsc.load_gather(
                    scb, [jnp.full((16,), k * WIN + r, jnp.int32)])
                for u in range(D // 16):
                    rb[r, pl.ds(u * 16, 16)] = rb[r, pl.ds(u * 16, 16)] * sv

            dout[k] = pltpu.async_copy(
                rb.at[pl.ds(0, wn)],
                out_hbm.at[pl.ds(base + k * WIN, wn)], osems[k % 2])
            din = nxt
        for k in range(max(0, nwin - 2), nwin):
            dout[k].wait()

    @pl.when(w < 31)
    def _():
        _windows(RPW)

    @pl.when(w == 31)
    def _():
        _windows(K - 31 * RPW)


def _gather_scale(x, idx, top_scores):
    return pl.kernel(
        _gather_body,
        out_type=jax.ShapeDtypeStruct((K, D), jnp.float32),
        mesh=plsc.VectorSubcoreMesh(core_axis_name="c", subcore_axis_name="s"),
        compiler_params=pltpu.CompilerParams(needs_layout_passes=False),
        scratch_types=(
            pltpu.VMEM((RPW,), jnp.int32),
            pltpu.VMEM((RPW,), jnp.float32),
            pltpu.VMEM((WIN, D), jnp.float32),
            pltpu.VMEM((WIN, D), jnp.float32),
            pltpu.SemaphoreType.DMA,
            pltpu.SemaphoreType.DMA,
            pltpu.SemaphoreType.DMA,
            pltpu.SemaphoreType.DMA,
        ),
    )(x, idx, top_scores)


def kernel(x, W1, b1, W2, b2):
    scores, keys = _scores(x, W1, b1, W2, b2)
    idx, keys_sorted = _sort(keys)
    top_scores = lax.bitcast_convert_type(KEY_TOP - keys_sorted, jnp.float32)
    scaled_feat = _gather_scale(x, idx, top_scores)
    return scaled_feat, idx, scores
